# baseline (speedup 1.0000x reference)
import functools

import jax
import jax.numpy as jnp
from jax import lax
from jax.experimental import pallas as pl
from jax.experimental.pallas import tpu as pltpu

_T = 16384
_DIM = 4096
_E = 64
_K = 8
_G = 8
_GS = _E // _G
_TG = 4
_SCALE = 2.5
_BT = 1024


def _gate_body(x_ref, w_ref, b_ref, wout_ref, iout_ref):
    x = x_ref[...]
    w = w_ref[...]
    s = lax.dot_general(w, x, (((1,), (1,)), ((), ())),
                        preferred_element_type=jnp.float32)
    neg_inf = jnp.float32(-jnp.inf)

    m = jnp.max(s, axis=0, keepdims=True)
    e = jnp.exp(s - m)
    probs = e / jnp.sum(e, axis=0, keepdims=True)
    biased = probs + b_ref[...]

    gscore_rows = []
    for g in range(_G):
        grp = biased[g * _GS:(g + 1) * _GS, :]
        m1 = jnp.max(grp, axis=0, keepdims=True)
        eq = grp == m1
        cnt = jnp.sum(eq.astype(jnp.float32), axis=0, keepdims=True)
        m2 = jnp.max(jnp.where(eq, neg_inf, grp), axis=0, keepdims=True)
        m2 = jnp.where(cnt > 1.5, m1, m2)
        gscore_rows.append(m1 + m2)
    gscore = jnp.concatenate(gscore_rows, axis=0)

    riota_g = lax.broadcasted_iota(jnp.int32, (_G, _BT), 0)
    sel = jnp.zeros((_G, _BT), dtype=jnp.bool_)
    gs = gscore
    for _ in range(_TG):
        mx = jnp.max(gs, axis=0, keepdims=True)
        a = jnp.min(jnp.where(gs == mx, riota_g, _E), axis=0, keepdims=True)
        hit = riota_g == a
        sel = sel | hit
        gs = jnp.where(hit, neg_inf, gs)

    ms_rows = []
    for g in range(_G):
        grp = biased[g * _GS:(g + 1) * _GS, :]
        ms_rows.append(jnp.where(sel[g:g + 1, :], grp, neg_inf))
    ms = jnp.concatenate(ms_rows, axis=0)

    riota_e = lax.broadcasted_iota(jnp.int32, (_E, _BT), 0)
    idx_rows = []
    w_rows = []
    for _ in range(_K):
        mx = jnp.max(ms, axis=0, keepdims=True)
        a = jnp.min(jnp.where(ms == mx, riota_e, _E), axis=0, keepdims=True)
        hit = riota_e == a
        wv = jnp.sum(jnp.where(hit, probs, 0.0), axis=0, keepdims=True)
        idx_rows.append(a)
        w_rows.append(wv)
        ms = jnp.where(hit, neg_inf, ms)

    wcat = jnp.concatenate(w_rows, axis=0) * jnp.float32(_SCALE)
    icat = jnp.concatenate(idx_rows, axis=0)
    wout_ref[...] = wcat.T
    iout_ref[...] = icat.T


@jax.jit
def kernel(x, weight, bias):
    bias2 = bias.reshape(_E, 1)
    grid = (_T // _BT,)
    wt, it = pl.pallas_call(
        _gate_body,
        grid=grid,
        in_specs=[
            pl.BlockSpec((_BT, _DIM), lambda i: (i, 0)),
            pl.BlockSpec((_E, _DIM), lambda i: (0, 0)),
            pl.BlockSpec((_E, 1), lambda i: (0, 0)),
        ],
        out_specs=[
            pl.BlockSpec((_BT, _K), lambda i: (i, 0)),
            pl.BlockSpec((_BT, _K), lambda i: (i, 0)),
        ],
        out_shape=[
            jax.ShapeDtypeStruct((_T, _K), jnp.float32),
            jax.ShapeDtypeStruct((_T, _K), jnp.int32),
        ],
        compiler_params=pltpu.CompilerParams(
            dimension_semantics=("parallel",),
            vmem_limit_bytes=100 * 1024 * 1024,
        ),
    )(x, weight, bias2)
    return wt.astype(x.dtype), it


# speedup vs baseline: 1.2161x; 1.2161x over previous
import functools

import jax
import jax.numpy as jnp
from jax import lax
from jax.experimental import pallas as pl
from jax.experimental.pallas import tpu as pltpu

_T = 16384
_DIM = 4096
_E = 64
_K = 8
_G = 8
_GS = _E // _G
_TG = 4
_SCALE = 2.5
_BT = 1024


def _gate_body(x_ref, w_ref, b_ref, wout_ref, iout_ref):
    x = x_ref[...]
    w = w_ref[...]
    s = lax.dot_general(w, x, (((1,), (1,)), ((), ())),
                        preferred_element_type=jnp.float32)
    neg_inf = jnp.float32(-jnp.inf)

    m = jnp.max(s, axis=0, keepdims=True)
    e = jnp.exp(s - m)
    probs = e / jnp.sum(e, axis=0, keepdims=True)
    biased = probs + b_ref[...]

    gscore_rows = []
    for g in range(_G):
        grp = biased[g * _GS:(g + 1) * _GS, :]
        m1 = jnp.max(grp, axis=0, keepdims=True)
        eq = grp == m1
        cnt = jnp.sum(eq.astype(jnp.float32), axis=0, keepdims=True)
        m2 = jnp.max(jnp.where(eq, neg_inf, grp), axis=0, keepdims=True)
        m2 = jnp.where(cnt > 1.5, m1, m2)
        gscore_rows.append(m1 + m2)
    gscore = jnp.concatenate(gscore_rows, axis=0)

    riota_g = lax.broadcasted_iota(jnp.int32, (_G, _BT), 0)
    sel = jnp.zeros((_G, _BT), dtype=jnp.bool_)
    gs = gscore
    for _ in range(_TG):
        mx = jnp.max(gs, axis=0, keepdims=True)
        a = jnp.min(jnp.where(gs == mx, riota_g, _E), axis=0, keepdims=True)
        hit = riota_g == a
        sel = sel | hit
        gs = jnp.where(hit, neg_inf, gs)

    ms_rows = []
    for g in range(_G):
        grp = biased[g * _GS:(g + 1) * _GS, :]
        ms_rows.append(jnp.where(sel[g:g + 1, :], grp, neg_inf))
    ms = jnp.concatenate(ms_rows, axis=0)

    riota_e = lax.broadcasted_iota(jnp.int32, (_E, _BT), 0)
    idx_rows = []
    w_rows = []
    for _ in range(_K):
        mx = jnp.max(ms, axis=0, keepdims=True)
        a = jnp.min(jnp.where(ms == mx, riota_e, _E), axis=0, keepdims=True)
        hit = riota_e == a
        wv = jnp.sum(jnp.where(hit, probs, 0.0), axis=0, keepdims=True)
        idx_rows.append(a)
        w_rows.append(wv)
        ms = jnp.where(hit, neg_inf, ms)

    wout_ref[...] = jnp.concatenate(w_rows, axis=0) * jnp.float32(_SCALE)
    iout_ref[...] = jnp.concatenate(idx_rows, axis=0)


@jax.jit
def kernel(x, weight, bias):
    bias2 = bias.reshape(_E, 1)
    grid = (_T // _BT,)
    wt, it = pl.pallas_call(
        _gate_body,
        grid=grid,
        in_specs=[
            pl.BlockSpec((_BT, _DIM), lambda i: (i, 0)),
            pl.BlockSpec((_E, _DIM), lambda i: (0, 0)),
            pl.BlockSpec((_E, 1), lambda i: (0, 0)),
        ],
        out_specs=[
            pl.BlockSpec((_K, _BT), lambda i: (0, i)),
            pl.BlockSpec((_K, _BT), lambda i: (0, i)),
        ],
        out_shape=[
            jax.ShapeDtypeStruct((_K, _T), jnp.float32),
            jax.ShapeDtypeStruct((_K, _T), jnp.int32),
        ],
        compiler_params=pltpu.CompilerParams(
            dimension_semantics=("parallel",),
            vmem_limit_bytes=100 * 1024 * 1024,
        ),
    )(x, weight, bias2)
    return wt.T.astype(x.dtype), it.T
